# Initial kernel scaffold; baseline (speedup 1.0000x reference)
import jax
import jax.numpy as jnp
from jax.experimental import pallas as pl


def kernel(x, start_pos, freqs_complex, k_cache, v_cache, wq, wk, wv, wo):
    raise NotImplementedError("write your pallas kernel here")


# speedup vs baseline: 4.7293x; 4.7293x over previous
import functools
import math

import jax
import jax.numpy as jnp
from jax.experimental import pallas as pl
from jax.experimental.pallas import tpu as pltpu

B = 16
D = 4096
H = 32
KVH = 8
HD = 128
N_REP = H // KVH
KV = 2048
QCHUNK = H * HD // 8
KCHUNK = KVH * HD // 8
OCHUNK = D // 8


def _pair_swap(x):
    n = x.shape[-1]
    left = pltpu.roll(x, n - 1, axis=len(x.shape) - 1)
    right = pltpu.roll(x, 1, axis=len(x.shape) - 1)
    lane = jax.lax.broadcasted_iota(jnp.int32, x.shape, len(x.shape) - 1)
    return jnp.where(lane % 2 == 0, left, right)


def _qkv_kernel(x_ref, wq_ref, wk_ref, wv_ref, cq_ref, sq_ref, ck_ref, sk_ref,
                q_ref, k_ref, v_ref):
    x = x_ref[...]
    q = jnp.dot(x, wq_ref[...], preferred_element_type=jnp.float32)
    k = jnp.dot(x, wk_ref[...], preferred_element_type=jnp.float32)
    v = jnp.dot(x, wv_ref[...], preferred_element_type=jnp.float32)
    q = q * cq_ref[...] + _pair_swap(q) * sq_ref[...]
    k = k * ck_ref[...] + _pair_swap(k) * sk_ref[...]
    q_ref[...] = q * (1.0 / math.sqrt(HD))
    k_ref[...] = k
    v_ref[...] = v


def _attn_kernel(q_ref, kc_ref, vc_ref, kn_ref, vn_ref, o_ref):
    q = q_ref[0, 0]
    kc = kc_ref[0, 0]
    vc = vc_ref[0, 0]
    kn = kn_ref[0, 0]
    vn = vn_ref[0, 0]
    dn = (((1,), (1,)), ((), ()))
    s = jax.lax.dot_general(q, kc, dn, preferred_element_type=jnp.float32)
    sn = jax.lax.dot_general(q, kn, dn, preferred_element_type=jnp.float32)
    m = jnp.maximum(jnp.max(s, axis=1, keepdims=True), sn)
    p = jnp.exp(s - m)
    pn = jnp.exp(sn - m)
    denom = jnp.sum(p, axis=1, keepdims=True) + pn
    dn2 = (((1,), (0,)), ((), ()))
    o = jax.lax.dot_general(p, vc, dn2, preferred_element_type=jnp.float32)
    o = (o + pn * vn) / denom
    o_ref[0, 0] = o


def _wo_kernel(x_ref, wo_ref, o_ref):
    o_ref[...] = jnp.dot(x_ref[...], wo_ref[...],
                         preferred_element_type=jnp.float32)


def kernel(x, start_pos, freqs_complex, k_cache, v_cache, wq, wk, wv, wo):
    del start_pos
    xf = x.reshape(B, D)
    cos = freqs_complex[0, :, 0]
    sin = freqs_complex[0, :, 1]
    cos2 = jnp.repeat(cos, 2)
    sgn2 = jnp.stack([-sin, sin], axis=-1).reshape(HD)
    cq = jnp.tile(cos2, QCHUNK // HD)[None, :]
    sq = jnp.tile(sgn2, QCHUNK // HD)[None, :]
    ck = cos2[None, :]
    sk = sgn2[None, :]

    q_rot, k_rot, v_new = pl.pallas_call(
        _qkv_kernel,
        grid=(8,),
        in_specs=[
            pl.BlockSpec((B, D), lambda i: (0, 0)),
            pl.BlockSpec((D, QCHUNK), lambda i: (0, i)),
            pl.BlockSpec((D, KCHUNK), lambda i: (0, i)),
            pl.BlockSpec((D, KCHUNK), lambda i: (0, i)),
            pl.BlockSpec((1, QCHUNK), lambda i: (0, 0)),
            pl.BlockSpec((1, QCHUNK), lambda i: (0, 0)),
            pl.BlockSpec((1, KCHUNK), lambda i: (0, 0)),
            pl.BlockSpec((1, KCHUNK), lambda i: (0, 0)),
        ],
        out_specs=[
            pl.BlockSpec((B, QCHUNK), lambda i: (0, i)),
            pl.BlockSpec((B, KCHUNK), lambda i: (0, i)),
            pl.BlockSpec((B, KCHUNK), lambda i: (0, i)),
        ],
        out_shape=[
            jax.ShapeDtypeStruct((B, H * HD), jnp.float32),
            jax.ShapeDtypeStruct((B, KVH * HD), jnp.float32),
            jax.ShapeDtypeStruct((B, KVH * HD), jnp.float32),
        ],
    )(xf, wq, wk, wv, cq, sq, ck, sk)

    qg = q_rot.reshape(B, KVH, N_REP, HD)
    kg = k_rot.reshape(B, KVH, 1, HD)
    vg = v_new.reshape(B, KVH, 1, HD)

    attn = pl.pallas_call(
        _attn_kernel,
        grid=(B, KVH),
        in_specs=[
            pl.BlockSpec((1, 1, N_REP, HD), lambda b, g: (b, g, 0, 0)),
            pl.BlockSpec((1, 1, KV, HD), lambda b, g: (b, g, 0, 0)),
            pl.BlockSpec((1, 1, KV, HD), lambda b, g: (b, g, 0, 0)),
            pl.BlockSpec((1, 1, 1, HD), lambda b, g: (b, g, 0, 0)),
            pl.BlockSpec((1, 1, 1, HD), lambda b, g: (b, g, 0, 0)),
        ],
        out_specs=pl.BlockSpec((1, 1, N_REP, HD), lambda b, g: (b, g, 0, 0)),
        out_shape=jax.ShapeDtypeStruct((B, KVH, N_REP, HD), jnp.float32),
    )(qg, k_cache, v_cache, kg, vg)

    attn_f = attn.reshape(B, H * HD)

    out = pl.pallas_call(
        _wo_kernel,
        grid=(8,),
        in_specs=[
            pl.BlockSpec((B, H * HD), lambda i: (0, 0)),
            pl.BlockSpec((H * HD, OCHUNK), lambda i: (0, i)),
        ],
        out_specs=pl.BlockSpec((B, OCHUNK), lambda i: (0, i)),
        out_shape=jax.ShapeDtypeStruct((B, D), jnp.float32),
    )(attn_f, wo)

    return out.reshape(B, 1, D)
